# baseline (speedup 1.0000x reference)
import functools

import jax
import jax.numpy as jnp
from jax import lax
from jax.experimental import pallas as pl
from jax.experimental.pallas import tpu as pltpu
from jax.experimental.pallas import tpu_sc as plsc

N = 10000
E = 320000
D = 128
H = 128
OUT = 128

NC = 2
NS = 16
NW = NC * NS
EW = E // NW
CHUNK = 80
NCHUNK = EW // CHUNK
GCHUNK = 80
NGCHUNK = EW // GCHUNK
NP = 10240
ROWS_PER_TILE = NP // NS

_f32 = jnp.float32
_i32 = jnp.int32

_sc_mesh = plsc.VectorSubcoreMesh(core_axis_name="c", subcore_axis_name="s")


_bf16 = jnp.bfloat16


def _proj_body(x_ref, wa_ref, wb_ref, u_ref, v_ref):
    xv = x_ref[...]
    u_ref[...] = jnp.dot(xv, wa_ref[...], preferred_element_type=_f32)
    v_ref[...] = jnp.dot(xv, wb_ref[...], preferred_element_type=_f32)


def _proj(x, wa, wb):
    bn = 1000
    return pl.pallas_call(
        _proj_body,
        grid=(N // bn,),
        in_specs=[
            pl.BlockSpec((bn, D), lambda i: (i, 0)),
            pl.BlockSpec((D, H), lambda i: (0, 0)),
            pl.BlockSpec((D, H), lambda i: (0, 0)),
        ],
        out_specs=[
            pl.BlockSpec((bn, H), lambda i: (i, 0)),
            pl.BlockSpec((bn, H), lambda i: (i, 0)),
        ],
        out_shape=[
            jax.ShapeDtypeStruct((N, H), _f32),
            jax.ShapeDtypeStruct((N, H), _f32),
        ],
    )(x, wa, wb)


EH = E // 2
EWH = EH // NW
GCH = 40
NGCH = EWH // GCH


@functools.partial(
    pl.kernel,
    out_type=jax.ShapeDtypeStruct((E,), _f32),
    mesh=_sc_mesh,
    scratch_types=[
        pltpu.VMEM((EW,), _i32),
        pltpu.VMEM((EW,), _i32),
    ],
    compiler_params=pltpu.CompilerParams(needs_layout_passes=False),
)
def _sc_d4(src_hbm, dst_hbm, pos_hbm, d4_hbm, idx_s, idx_d):
    wid = lax.axis_index("s") * NC + lax.axis_index("c")
    base = wid * EW
    pltpu.sync_copy(src_hbm.at[pl.ds(base, EW)], idx_s)
    pltpu.sync_copy(dst_hbm.at[pl.ds(base, EW)], idx_d)

    def _phase1(pos_v, d4_v):
        pltpu.sync_copy(pos_hbm, pos_v)

        @pl.loop(0, EW // 16)
        def _d4_step(j):
            rs = idx_s[pl.ds(j * 16, 16)]
            rd = idx_d[pl.ds(j * 16, 16)]
            rs3 = rs * 3
            rd3 = rd * 3
            acc = jnp.zeros((16,), _f32)
            for c in range(3):
                cc = jnp.full((16,), c, _i32)
                df = (plsc.load_gather(pos_v, [rs3 + cc])
                      - plsc.load_gather(pos_v, [rd3 + cc]))
                d2 = df * df
                acc = acc + d2 * d2
            d4_v[pl.ds(j * 16, 16)] = acc

        pltpu.sync_copy(d4_v, d4_hbm.at[pl.ds(base, EW)])

    pl.run_scoped(_phase1, pltpu.VMEM((N * 3,), _f32),
                  pltpu.VMEM((EW,), _f32))


@functools.partial(
    pl.kernel,
    out_type=jax.ShapeDtypeStruct((EH, H), _f32),
    mesh=_sc_mesh,
    scratch_types=[
        pltpu.VMEM((EWH,), _i32),
        pltpu.VMEM((EWH,), _i32),
        pltpu.SemaphoreType.DMA,
        pltpu.SemaphoreType.DMA,
    ],
    compiler_params=pltpu.CompilerParams(needs_layout_passes=False),
)
def _sc_gather(u_hbm, v_hbm, src_hbm, dst_hbm, g_hbm, idx_s, idx_d,
               sem0, sem1):
    wid = lax.axis_index("s") * NC + lax.axis_index("c")
    base = wid * EWH
    pltpu.sync_copy(src_hbm.at[pl.ds(base, EWH)], idx_s)
    pltpu.sync_copy(dst_hbm.at[pl.ds(base, EWH)], idx_d)

    def _phase2(a0, b0, a1, b1):
        bufs = ((a0, b0, sem0), (a1, b1, sem1))

        def _start(k, slot):
            a, b, s = bufs[slot]
            off = k * GCH
            pltpu.async_copy(u_hbm.at[idx_s.at[pl.ds(off, GCH)]], a, s)
            pltpu.async_copy(v_hbm.at[idx_d.at[pl.ds(off, GCH)]], b, s)

        def _wait(k, slot):
            a, b, s = bufs[slot]
            off = k * GCH
            pltpu.make_async_copy(
                u_hbm.at[idx_s.at[pl.ds(off, GCH)]], a, s).wait()
            pltpu.make_async_copy(
                v_hbm.at[idx_d.at[pl.ds(off, GCH)]], b, s).wait()

        _start(0, 0)

        @pl.loop(0, NGCH, step=2)
        def _outer(k):
            for j in range(2):
                kj = k + j
                a, b, s = bufs[j]

                @pl.when(kj < NGCH)
                def _():
                    @pl.when(kj + 1 < NGCH)
                    def _():
                        _start(kj + 1, 1 - j)

                    _wait(kj, j)

                    @pl.loop(0, GCH)
                    def _row(r):
                        for c in range(H // 16):
                            sl = pl.ds(c * 16, 16)
                            plsc.addupdate(a.at[r, sl], b[r, sl])

                    pltpu.sync_copy(
                        a, g_hbm.at[pl.ds(base + kj * GCH, GCH)])

    pl.run_scoped(_phase2, *([pltpu.VMEM((GCH, H), _f32)] * 4))


_tanh = jnp.tanh


def _edge_body(g_ref, d4_ref, mask_ref, wco_ref, b1_ref, w2_ref, b2_ref,
               wi_ref, bi_ref, o_ref):
    dd = jnp.sqrt(d4_ref[0])
    nb = dd.shape[0]
    db = jnp.broadcast_to(dd[:, None, :], (nb, 128, 128)).reshape(nb * 128, 128)
    dsel = db * mask_ref[...]
    pre = (g_ref[...]
           + jnp.dot(dsel, wco_ref[...], preferred_element_type=_f32)
           + b1_ref[...])
    h1 = _tanh(pre)
    m = _tanh(jnp.dot(h1, w2_ref[...], preferred_element_type=_f32)
              + b2_ref[...])
    e = _tanh(jnp.dot(m, wi_ref[...], preferred_element_type=_f32)
              + bi_ref[...])
    o_ref[...] = e * m


def _edge_mlp(g, d4, mask, wco, b1, w2, b2, wi, bi):
    be = 1280
    return pl.pallas_call(
        _edge_body,
        grid=(EH // be,),
        in_specs=[
            pl.BlockSpec((be, H), lambda i: (i, 0)),
            pl.BlockSpec((1, be // 128, 128), lambda i: (i, 0, 0)),
            pl.BlockSpec((be, 128), lambda i: (0, 0)),
            pl.BlockSpec((128, H), lambda i: (0, 0)),
            pl.BlockSpec((1, H), lambda i: (0, 0)),
            pl.BlockSpec((H, H), lambda i: (0, 0)),
            pl.BlockSpec((1, H), lambda i: (0, 0)),
            pl.BlockSpec((H, 1), lambda i: (0, 0)),
            pl.BlockSpec((1, 1), lambda i: (0, 0)),
        ],
        out_specs=pl.BlockSpec((be, H), lambda i: (i, 0)),
        out_shape=jax.ShapeDtypeStruct((EH, H), _f32),
    )(g, d4, mask, wco, b1, w2, b2, wi, bi)


SCH = 40
NSCH = EWH // SCH


def _sc_scatter_body(w_hbm, src_hbm, out_hbm, sem0, sem1, acc):
    c = lax.axis_index("c")
    s = lax.axis_index("s")
    base = (s * NC + c) * EWH
    rows0 = s * ROWS_PER_TILE

    def _zero(zbuf):
        @pl.loop(0, 128)
        def _zrow(r):
            for cc in range(H // 16):
                zbuf[r, pl.ds(cc * 16, 16)] = jnp.zeros((16,), _f32)

        for t in range(ROWS_PER_TILE // 128):
            pltpu.sync_copy(zbuf, acc.at[pl.ds(rows0 + t * 128, 128)])

    pl.run_scoped(_zero, pltpu.VMEM((128, H), _f32))
    plsc.subcore_barrier()

    def _phase(idx0, w0, idx1, w1):
        pairs = ((idx0, w0, sem0), (idx1, w1, sem1))

        def _start(k, slot):
            i, w, sm = pairs[slot]
            off = base + k * SCH
            pltpu.async_copy(src_hbm.at[pl.ds(off, SCH)], i, sm)
            pltpu.async_copy(w_hbm.at[pl.ds(off, SCH)], w, sm)

        def _wait(k, slot):
            i, w, sm = pairs[slot]
            off = base + k * SCH
            pltpu.make_async_copy(src_hbm.at[pl.ds(off, SCH)], i, sm).wait()
            pltpu.make_async_copy(w_hbm.at[pl.ds(off, SCH)], w, sm).wait()

        _start(0, 0)

        @pl.loop(0, NSCH, step=2)
        def _outer(k):
            for j in range(2):
                kj = k + j
                i, w, sm = pairs[j]

                @pl.when(kj < NSCH)
                def _():
                    @pl.when(kj + 1 < NSCH)
                    def _():
                        _start(kj + 1, 1 - j)

                    _wait(kj, j)
                    pltpu.sync_copy(w, acc.at[i], add=True)

    pl.run_scoped(_phase, pltpu.VMEM((SCH,), _i32),
                  pltpu.VMEM((SCH, H), _f32), pltpu.VMEM((SCH,), _i32),
                  pltpu.VMEM((SCH, H), _f32))
    plsc.subcore_barrier()

    @pl.loop(0, ROWS_PER_TILE // 128)
    def _wb(t):
        r0 = rows0 + t * 128
        pltpu.sync_copy(acc.at[pl.ds(r0, 128)], out_hbm.at[c, pl.ds(r0, 128)])


@functools.partial(
    pl.kernel,
    out_type=jax.ShapeDtypeStruct((NC, NP, H), _f32),
    mesh=_sc_mesh,
    scratch_types=[
        pltpu.SemaphoreType.DMA,
        pltpu.SemaphoreType.DMA,
        pltpu.VMEM_SHARED((NP, H), _f32),
    ],
    compiler_params=pltpu.CompilerParams(needs_layout_passes=False),
)
def _sc_scatter(w_hbm, src_hbm, out_hbm, sem0, sem1, acc):
    _sc_scatter_body(w_hbm, src_hbm, out_hbm, sem0, sem1, acc)


def _head_body(x_ref, p0_ref, p1_ref, p2_ref, p3_ref, fw1a_ref, fw1b_ref,
               fb1_ref, fw2_ref, fb2_ref, w1_ref, b1_ref, w2_ref, b2_ref,
               w3_ref, b3_ref, w4_ref, b4_ref, w5_ref, b5_ref, o_ref):
    xv = x_ref[...]
    mi = ((p0_ref[0] + p1_ref[0]) + (p2_ref[0] + p3_ref[0]))
    h = _tanh(jnp.dot(xv, fw1a_ref[...], preferred_element_type=_f32)
              + jnp.dot(mi, fw1b_ref[...], preferred_element_type=_f32)
              + fb1_ref[...])
    h = jnp.dot(h, fw2_ref[...], preferred_element_type=_f32) + fb2_ref[...]
    t = xv + h

    def lrelu(z):
        return jnp.where(z >= 0, z, 0.01 * z)

    t = lrelu(jnp.dot(t, w1_ref[...], preferred_element_type=_f32) + b1_ref[...])
    t = lrelu(jnp.dot(t, w2_ref[...], preferred_element_type=_f32) + b2_ref[...])
    t = lrelu(jnp.dot(t, w3_ref[...], preferred_element_type=_f32) + b3_ref[...])
    t = lrelu(jnp.dot(t, w4_ref[...], preferred_element_type=_f32) + b4_ref[...])
    o_ref[...] = (jnp.dot(t, w5_ref[...], preferred_element_type=_f32)
                  + b5_ref[...])


def _head(x, pa, pb, fw1a, fw1b, fb1, fw2, fb2, w1, b1, w2, b2, w3, b3,
          w4, b4, w5, b5):
    bn = 1000
    mat = lambda: pl.BlockSpec((H, H), lambda i: (0, 0))
    vec = lambda: pl.BlockSpec((1, H), lambda i: (0, 0))
    blk = lambda: pl.BlockSpec((bn, H), lambda i: (i, 0))
    part = lambda c: pl.BlockSpec((1, bn, H), lambda i, c=c: (c, i, 0))
    return pl.pallas_call(
        _head_body,
        grid=(N // bn,),
        in_specs=[
            blk(), part(0), part(1), part(0), part(1),
            mat(), mat(), vec(), mat(), vec(),
            mat(), vec(), mat(), vec(), mat(), vec(), mat(), vec(),
            pl.BlockSpec((H, OUT), lambda i: (0, 0)),
            pl.BlockSpec((1, OUT), lambda i: (0, 0)),
        ],
        out_specs=pl.BlockSpec((bn, OUT), lambda i: (i, 0)),
        out_shape=jax.ShapeDtypeStruct((N, OUT), _f32),
    )(x, pa, pa, pb, pb, fw1a, fw1b, fb1, fw2, fb2, w1, b1, w2, b2, w3, b3,
      w4, b4, w5, b5)


def kernel(x, pos, edge_index, fe_w1, fe_b1, fe_w2, fe_b2, finf_w, finf_b,
           fh_w1, fh_b1, fh_w2, fh_b2, w1, b1, w2, b2, w3, b3, w4, b4, w5, b5):
    src = edge_index[0]
    dst = edge_index[1]
    wa = fe_w1[:D]
    wb = fe_w1[D:2 * D]
    wc = fe_w1[2 * D:2 * D + 1]

    d4 = _sc_d4(src, dst, pos.reshape(-1))
    u, v = _proj(x, wa, wb)
    be = 1280
    mask = (jax.lax.broadcasted_iota(_i32, (be, 128), 0) % 128
            == jax.lax.broadcasted_iota(_i32, (be, 128), 1)).astype(_f32)
    wco = jnp.ones((128, 1), _f32) * wc
    b1r = fe_b1.reshape(1, H)
    b2r = fe_b2.reshape(1, H)
    bir = finf_b.reshape(1, 1)
    d4h = d4.reshape(2, EH // be, be // 128, 128)

    g0 = _sc_gather(u, v, src[:EH], dst[:EH])
    w0 = _edge_mlp(g0, d4h[0], mask, wco, b1r, fe_w2, b2r, finf_w, bir)
    g1 = _sc_gather(u, v, src[EH:], dst[EH:])
    w1_ = _edge_mlp(g1, d4h[1], mask, wco, b1r, fe_w2, b2r, finf_w, bir)
    pa = _sc_scatter(w0, src[:EH])
    pb = _sc_scatter(w1_, src[EH:])

    out = _head(x, pa, pb, fh_w1[:D], fh_w1[D:], fh_b1.reshape(1, H),
                fh_w2, fh_b2.reshape(1, H), w1, b1.reshape(1, H), w2,
                b2.reshape(1, H), w3, b3.reshape(1, H), w4, b4.reshape(1, H),
                w5, b5.reshape(1, OUT))
    return out


# speedup vs baseline: 1.0364x; 1.0364x over previous
import functools

import jax
import jax.numpy as jnp
from jax import lax
from jax.experimental import pallas as pl
from jax.experimental.pallas import tpu as pltpu
from jax.experimental.pallas import tpu_sc as plsc

N = 10000
E = 320000
D = 128
H = 128
OUT = 128

NC = 2
NS = 16
NW = NC * NS
EW = E // NW
CHUNK = 80
NCHUNK = EW // CHUNK
GCHUNK = 80
NGCHUNK = EW // GCHUNK
NP = 10240
ROWS_PER_TILE = NP // NS

_f32 = jnp.float32
_i32 = jnp.int32

_sc_mesh = plsc.VectorSubcoreMesh(core_axis_name="c", subcore_axis_name="s")


_bf16 = jnp.bfloat16


def _proj_body(x_ref, wa_ref, wb_ref, u_ref, v_ref):
    xv = x_ref[...]
    u_ref[...] = jnp.dot(xv, wa_ref[...], preferred_element_type=_f32)
    v_ref[...] = jnp.dot(xv, wb_ref[...], preferred_element_type=_f32)


def _proj(x, wa, wb):
    bn = 1000
    return pl.pallas_call(
        _proj_body,
        grid=(N // bn,),
        in_specs=[
            pl.BlockSpec((bn, D), lambda i: (i, 0)),
            pl.BlockSpec((D, H), lambda i: (0, 0)),
            pl.BlockSpec((D, H), lambda i: (0, 0)),
        ],
        out_specs=[
            pl.BlockSpec((bn, H), lambda i: (i, 0)),
            pl.BlockSpec((bn, H), lambda i: (i, 0)),
        ],
        out_shape=[
            jax.ShapeDtypeStruct((N, H), _f32),
            jax.ShapeDtypeStruct((N, H), _f32),
        ],
    )(x, wa, wb)


EH = E // 2
EWH = EH // NW
GCH = 40
NGCH = EWH // GCH


@functools.partial(
    pl.kernel,
    out_type=jax.ShapeDtypeStruct((E,), _f32),
    mesh=_sc_mesh,
    scratch_types=[
        pltpu.VMEM((EW,), _i32),
        pltpu.VMEM((EW,), _i32),
    ],
    compiler_params=pltpu.CompilerParams(needs_layout_passes=False),
)
def _sc_d4(src_hbm, dst_hbm, pos_hbm, d4_hbm, idx_s, idx_d):
    wid = lax.axis_index("s") * NC + lax.axis_index("c")
    base = wid * EW
    pltpu.sync_copy(src_hbm.at[pl.ds(base, EW)], idx_s)
    pltpu.sync_copy(dst_hbm.at[pl.ds(base, EW)], idx_d)

    def _phase1(pos_v, d4_v):
        pltpu.sync_copy(pos_hbm, pos_v)

        @pl.loop(0, EW // 16)
        def _d4_step(j):
            rs = idx_s[pl.ds(j * 16, 16)]
            rd = idx_d[pl.ds(j * 16, 16)]
            rs3 = rs * 3
            rd3 = rd * 3
            acc = jnp.zeros((16,), _f32)
            for c in range(3):
                cc = jnp.full((16,), c, _i32)
                df = (plsc.load_gather(pos_v, [rs3 + cc])
                      - plsc.load_gather(pos_v, [rd3 + cc]))
                d2 = df * df
                acc = acc + d2 * d2
            d4_v[pl.ds(j * 16, 16)] = acc

        pltpu.sync_copy(d4_v, d4_hbm.at[pl.ds(base, EW)])

    pl.run_scoped(_phase1, pltpu.VMEM((N * 3,), _f32),
                  pltpu.VMEM((EW,), _f32))


@functools.partial(
    pl.kernel,
    out_type=jax.ShapeDtypeStruct((EH, H), _f32),
    mesh=_sc_mesh,
    scratch_types=[
        pltpu.VMEM((EWH,), _i32),
        pltpu.VMEM((EWH,), _i32),
        pltpu.SemaphoreType.DMA,
        pltpu.SemaphoreType.DMA,
    ],
    compiler_params=pltpu.CompilerParams(needs_layout_passes=False),
)
def _sc_gather(u_hbm, v_hbm, src_hbm, dst_hbm, g_hbm, idx_s, idx_d,
               sem0, sem1):
    wid = lax.axis_index("s") * NC + lax.axis_index("c")
    base = wid * EWH
    pltpu.sync_copy(src_hbm.at[pl.ds(base, EWH)], idx_s)
    pltpu.sync_copy(dst_hbm.at[pl.ds(base, EWH)], idx_d)

    def _phase2(a0, b0, a1, b1):
        bufs = ((a0, b0, sem0), (a1, b1, sem1))

        def _start(k, slot):
            a, b, s = bufs[slot]
            off = k * GCH
            pltpu.async_copy(u_hbm.at[idx_s.at[pl.ds(off, GCH)]], a, s)
            pltpu.async_copy(v_hbm.at[idx_d.at[pl.ds(off, GCH)]], b, s)

        def _wait(k, slot):
            a, b, s = bufs[slot]
            off = k * GCH
            pltpu.make_async_copy(
                u_hbm.at[idx_s.at[pl.ds(off, GCH)]], a, s).wait()
            pltpu.make_async_copy(
                v_hbm.at[idx_d.at[pl.ds(off, GCH)]], b, s).wait()

        _start(0, 0)

        @pl.loop(0, NGCH, step=2)
        def _outer(k):
            for j in range(2):
                kj = k + j
                a, b, s = bufs[j]

                @pl.when(kj < NGCH)
                def _():
                    @pl.when(kj + 1 < NGCH)
                    def _():
                        _start(kj + 1, 1 - j)

                    _wait(kj, j)

                    @pl.loop(0, GCH)
                    def _row(r):
                        for c in range(H // 16):
                            sl = pl.ds(c * 16, 16)
                            plsc.addupdate(a.at[r, sl], b[r, sl])

                    pltpu.sync_copy(
                        a, g_hbm.at[pl.ds(base + kj * GCH, GCH)])

    pl.run_scoped(_phase2, *([pltpu.VMEM((GCH, H), _f32)] * 4))


_tanh = jnp.tanh


def _edge_body(g_ref, d4_ref, mask_ref, wco_ref, b1_ref, w2_ref, b2_ref,
               wi_ref, bi_ref, o_ref):
    dd = jnp.sqrt(d4_ref[0])
    nb = dd.shape[0]
    db = jnp.broadcast_to(dd[:, None, :], (nb, 128, 128)).reshape(nb * 128, 128)
    dsel = db * mask_ref[...]
    pre = (g_ref[...]
           + jnp.dot(dsel, wco_ref[...], preferred_element_type=_f32)
           + b1_ref[...])
    h1 = _tanh(pre)
    m = _tanh(jnp.dot(h1, w2_ref[...], preferred_element_type=_f32)
              + b2_ref[...])
    e = _tanh(jnp.dot(m, wi_ref[...], preferred_element_type=_f32)
              + bi_ref[...])
    o_ref[...] = e * m


def _edge_mlp(g, d4, mask, wco, b1, w2, b2, wi, bi):
    be = 6400
    return pl.pallas_call(
        _edge_body,
        grid=(EH // be,),
        in_specs=[
            pl.BlockSpec((be, H), lambda i: (i, 0)),
            pl.BlockSpec((1, be // 128, 128), lambda i: (i, 0, 0)),
            pl.BlockSpec((be, 128), lambda i: (0, 0)),
            pl.BlockSpec((128, H), lambda i: (0, 0)),
            pl.BlockSpec((1, H), lambda i: (0, 0)),
            pl.BlockSpec((H, H), lambda i: (0, 0)),
            pl.BlockSpec((1, H), lambda i: (0, 0)),
            pl.BlockSpec((H, 1), lambda i: (0, 0)),
            pl.BlockSpec((1, 1), lambda i: (0, 0)),
        ],
        out_specs=pl.BlockSpec((be, H), lambda i: (i, 0)),
        out_shape=jax.ShapeDtypeStruct((EH, H), _f32),
    )(g, d4, mask, wco, b1, w2, b2, wi, bi)


SCH = 40
NSCH = EWH // SCH


def _sc_scatter_body(w_hbm, src_hbm, out_hbm, sem0, sem1, acc):
    c = lax.axis_index("c")
    s = lax.axis_index("s")
    base = (s * NC + c) * EWH
    rows0 = s * ROWS_PER_TILE

    def _zero(zbuf):
        @pl.loop(0, 128)
        def _zrow(r):
            for cc in range(H // 16):
                zbuf[r, pl.ds(cc * 16, 16)] = jnp.zeros((16,), _f32)

        for t in range(ROWS_PER_TILE // 128):
            pltpu.sync_copy(zbuf, acc.at[pl.ds(rows0 + t * 128, 128)])

    pl.run_scoped(_zero, pltpu.VMEM((128, H), _f32))
    plsc.subcore_barrier()

    def _phase(idx0, w0, idx1, w1):
        pairs = ((idx0, w0, sem0), (idx1, w1, sem1))

        def _start(k, slot):
            i, w, sm = pairs[slot]
            off = base + k * SCH
            pltpu.async_copy(src_hbm.at[pl.ds(off, SCH)], i, sm)
            pltpu.async_copy(w_hbm.at[pl.ds(off, SCH)], w, sm)

        def _wait(k, slot):
            i, w, sm = pairs[slot]
            off = base + k * SCH
            pltpu.make_async_copy(src_hbm.at[pl.ds(off, SCH)], i, sm).wait()
            pltpu.make_async_copy(w_hbm.at[pl.ds(off, SCH)], w, sm).wait()

        _start(0, 0)

        @pl.loop(0, NSCH, step=2)
        def _outer(k):
            for j in range(2):
                kj = k + j
                i, w, sm = pairs[j]

                @pl.when(kj < NSCH)
                def _():
                    @pl.when(kj + 1 < NSCH)
                    def _():
                        _start(kj + 1, 1 - j)

                    _wait(kj, j)
                    pltpu.sync_copy(w, acc.at[i], add=True)

    pl.run_scoped(_phase, pltpu.VMEM((SCH,), _i32),
                  pltpu.VMEM((SCH, H), _f32), pltpu.VMEM((SCH,), _i32),
                  pltpu.VMEM((SCH, H), _f32))
    plsc.subcore_barrier()

    @pl.loop(0, ROWS_PER_TILE // 128)
    def _wb(t):
        r0 = rows0 + t * 128
        pltpu.sync_copy(acc.at[pl.ds(r0, 128)], out_hbm.at[c, pl.ds(r0, 128)])


@functools.partial(
    pl.kernel,
    out_type=jax.ShapeDtypeStruct((NC, NP, H), _f32),
    mesh=_sc_mesh,
    scratch_types=[
        pltpu.SemaphoreType.DMA,
        pltpu.SemaphoreType.DMA,
        pltpu.VMEM_SHARED((NP, H), _f32),
    ],
    compiler_params=pltpu.CompilerParams(needs_layout_passes=False),
)
def _sc_scatter(w_hbm, src_hbm, out_hbm, sem0, sem1, acc):
    _sc_scatter_body(w_hbm, src_hbm, out_hbm, sem0, sem1, acc)


def _head_body(x_ref, p0_ref, p1_ref, p2_ref, p3_ref, fw1a_ref, fw1b_ref,
               fb1_ref, fw2_ref, fb2_ref, w1_ref, b1_ref, w2_ref, b2_ref,
               w3_ref, b3_ref, w4_ref, b4_ref, w5_ref, b5_ref, o_ref):
    xv = x_ref[...]
    mi = ((p0_ref[0] + p1_ref[0]) + (p2_ref[0] + p3_ref[0]))
    h = _tanh(jnp.dot(xv, fw1a_ref[...], preferred_element_type=_f32)
              + jnp.dot(mi, fw1b_ref[...], preferred_element_type=_f32)
              + fb1_ref[...])
    h = jnp.dot(h, fw2_ref[...], preferred_element_type=_f32) + fb2_ref[...]
    t = xv + h

    def lrelu(z):
        return jnp.where(z >= 0, z, 0.01 * z)

    t = lrelu(jnp.dot(t, w1_ref[...], preferred_element_type=_f32) + b1_ref[...])
    t = lrelu(jnp.dot(t, w2_ref[...], preferred_element_type=_f32) + b2_ref[...])
    t = lrelu(jnp.dot(t, w3_ref[...], preferred_element_type=_f32) + b3_ref[...])
    t = lrelu(jnp.dot(t, w4_ref[...], preferred_element_type=_f32) + b4_ref[...])
    o_ref[...] = (jnp.dot(t, w5_ref[...], preferred_element_type=_f32)
                  + b5_ref[...])


def _head(x, pa, pb, fw1a, fw1b, fb1, fw2, fb2, w1, b1, w2, b2, w3, b3,
          w4, b4, w5, b5):
    bn = 1000
    mat = lambda: pl.BlockSpec((H, H), lambda i: (0, 0))
    vec = lambda: pl.BlockSpec((1, H), lambda i: (0, 0))
    blk = lambda: pl.BlockSpec((bn, H), lambda i: (i, 0))
    part = lambda c: pl.BlockSpec((1, bn, H), lambda i, c=c: (c, i, 0))
    return pl.pallas_call(
        _head_body,
        grid=(N // bn,),
        in_specs=[
            blk(), part(0), part(1), part(0), part(1),
            mat(), mat(), vec(), mat(), vec(),
            mat(), vec(), mat(), vec(), mat(), vec(), mat(), vec(),
            pl.BlockSpec((H, OUT), lambda i: (0, 0)),
            pl.BlockSpec((1, OUT), lambda i: (0, 0)),
        ],
        out_specs=pl.BlockSpec((bn, OUT), lambda i: (i, 0)),
        out_shape=jax.ShapeDtypeStruct((N, OUT), _f32),
    )(x, pa, pa, pb, pb, fw1a, fw1b, fb1, fw2, fb2, w1, b1, w2, b2, w3, b3,
      w4, b4, w5, b5)


def kernel(x, pos, edge_index, fe_w1, fe_b1, fe_w2, fe_b2, finf_w, finf_b,
           fh_w1, fh_b1, fh_w2, fh_b2, w1, b1, w2, b2, w3, b3, w4, b4, w5, b5):
    src = edge_index[0]
    dst = edge_index[1]
    wa = fe_w1[:D]
    wb = fe_w1[D:2 * D]
    wc = fe_w1[2 * D:2 * D + 1]

    d4 = _sc_d4(src, dst, pos.reshape(-1))
    u, v = _proj(x, wa, wb)
    be = 6400
    mask = (jax.lax.broadcasted_iota(_i32, (be, 128), 0) % 128
            == jax.lax.broadcasted_iota(_i32, (be, 128), 1)).astype(_f32)
    wco = jnp.ones((128, 1), _f32) * wc
    b1r = fe_b1.reshape(1, H)
    b2r = fe_b2.reshape(1, H)
    bir = finf_b.reshape(1, 1)
    d4h = d4.reshape(2, EH // be, be // 128, 128)

    g0 = _sc_gather(u, v, src[:EH], dst[:EH])
    w0 = _edge_mlp(g0, d4h[0], mask, wco, b1r, fe_w2, b2r, finf_w, bir)
    g1 = _sc_gather(u, v, src[EH:], dst[EH:])
    w1_ = _edge_mlp(g1, d4h[1], mask, wco, b1r, fe_w2, b2r, finf_w, bir)
    pa = _sc_scatter(w0, src[:EH])
    pb = _sc_scatter(w1_, src[EH:])

    out = _head(x, pa, pb, fh_w1[:D], fh_w1[D:], fh_b1.reshape(1, H),
                fh_w2, fh_b2.reshape(1, H), w1, b1.reshape(1, H), w2,
                b2.reshape(1, H), w3, b3.reshape(1, H), w4, b4.reshape(1, H),
                w5, b5.reshape(1, OUT))
    return out
